# baseline (speedup 1.0000x reference)
import functools

import jax
import jax.numpy as jnp
from jax import lax
from jax.experimental import pallas as pl
from jax.experimental.pallas import tpu as pltpu
from jax.experimental.pallas import tpu_sc as plsc

AFD = 128
GFD = 256
NBRF = 41
MNB = 16
NAT = 10000
NEDGE = NAT * MNB
NCRY = 16
KCRY = 625
KPAD = 640

_SC_CORES = 2
_SC_SUBCORES = 16
_NW = _SC_CORES * _SC_SUBCORES


def _sc_gather(table, idx_flat, chunk):
    B = idx_flat.shape[0]
    D = table.shape[1]
    cpw = B // (chunk * _NW)
    mesh = plsc.VectorSubcoreMesh(core_axis_name="c", subcore_axis_name="s")

    @functools.partial(
        pl.kernel,
        mesh=mesh,
        out_type=jax.ShapeDtypeStruct((B, D), table.dtype),
        compiler_params=pltpu.CompilerParams(use_tc_tiling_on_sc=False),
        scratch_types=[
            pltpu.VMEM((cpw * chunk,), jnp.int32),
            pltpu.VMEM((chunk, D), table.dtype),
            pltpu.VMEM((chunk, D), table.dtype),
            pltpu.SemaphoreType.DMA,
            pltpu.SemaphoreType.DMA,
        ],
    )
    def k(table_hbm, idx_hbm, out_hbm, idx_v, r0, r1, s0, s1):
        wid = lax.axis_index("s") * _SC_CORES + lax.axis_index("c")
        base = wid * cpw * chunk
        pltpu.sync_copy(idx_hbm.at[pl.ds(base, cpw * chunk)], idx_v)
        pltpu.async_copy(table_hbm.at[idx_v.at[pl.ds(0, chunk)]], r0, s0)

        def wait_rows(buf, sem):
            pltpu.make_async_copy(table_hbm.at[pl.ds(0, chunk)], buf, sem).wait()

        @pl.loop(0, cpw, step=2)
        def _(c):
            @pl.when(c + 1 < cpw)
            def _():
                pltpu.async_copy(
                    table_hbm.at[idx_v.at[pl.ds((c + 1) * chunk, chunk)]], r1, s1
                )

            wait_rows(r0, s0)
            pltpu.sync_copy(r0, out_hbm.at[pl.ds(base + c * chunk, chunk)])

            @pl.when(c + 2 < cpw)
            def _():
                pltpu.async_copy(
                    table_hbm.at[idx_v.at[pl.ds((c + 2) * chunk, chunk)]], r0, s0
                )

            @pl.when(c + 1 < cpw)
            def _():
                wait_rows(r1, s1)
                pltpu.sync_copy(
                    r1, out_hbm.at[pl.ds(base + (c + 1) * chunk, chunk)]
                )

    return k(table, idx_flat)


def _pack_table(a):
    ab = a.astype(jnp.bfloat16)
    lo = lax.bitcast_convert_type(ab[:, : AFD // 2], jnp.uint16).astype(jnp.uint32)
    hi = lax.bitcast_convert_type(ab[:, AFD // 2 :], jnp.uint16).astype(jnp.uint32)
    return lax.bitcast_convert_type(lo | (hi << 16), jnp.int32)


def _unpack(g32):
    lo = lax.bitcast_convert_type(g32 << 16, jnp.float32)
    hi = lax.bitcast_convert_type(g32 & jnp.int32(-65536), jnp.float32)
    return jnp.concatenate([lo, hi], axis=-1)


def _unpack_bf(g32):
    lo = lax.bitcast_convert_type(g32 << 16, jnp.float32).astype(jnp.bfloat16)
    hi = lax.bitcast_convert_type(
        g32 & jnp.int32(-65536), jnp.float32).astype(jnp.bfloat16)
    return jnp.concatenate([lo, hi], axis=-1)


def _embed(x, W, b):
    N, F = x.shape
    TO = W.shape[1]
    TN = 2000

    def body(x_ref, w_ref, b_ref, o_ref):
        o_ref[...] = (
            jnp.dot(x_ref[...], w_ref[...], preferred_element_type=jnp.float32)
            + b_ref[...]
        )

    return pl.pallas_call(
        body,
        out_shape=jax.ShapeDtypeStruct((N, TO), jnp.float32),
        grid=(N // TN,),
        in_specs=[
            pl.BlockSpec((TN, F), lambda i: (i, 0)),
            pl.BlockSpec((F, TO), lambda i: (0, 0)),
            pl.BlockSpec((1, TO), lambda i: (0, 0)),
        ],
        out_specs=pl.BlockSpec((TN, TO), lambda i: (i, 0)),
    )(x, W, b.reshape(1, TO))


_T1 = 1000


def _conv_layer(a, G3, nbrT, Wse, Wan, Wnbr, g1, b1, g2, b2):
    grid = (3, NAT // _T1)

    def body(a_ref, g_ref, n_ref, wse_ref, wan_ref, wnbr_ref, g1_ref, b1_ref,
             g2_ref, b2_ref, o_ref, pk_ref, s_sc, st1, st2):
        ph = pl.program_id(0)
        i = pl.program_id(1)

        @pl.when(ph == 0)
        def _():
            wan_bf = wan_ref[...].astype(jnp.bfloat16)
            wnbr_bf = wnbr_ref[...].astype(jnp.bfloat16)
            wse_bf = wse_ref[...].astype(jnp.bfloat16)
            u = jnp.dot(a_ref[...].astype(jnp.bfloat16), wse_bf,
                        preferred_element_type=jnp.float32)
            acc_p = None
            acc_sq = None
            for m in range(MNB):
                ge = _unpack_bf(g_ref[m])
                p = jnp.dot(ge, wan_bf, preferred_element_type=jnp.float32)
                p = p + jnp.dot(n_ref[m], wnbr_bf,
                                preferred_element_type=jnp.float32)
                acc_p = p if acc_p is None else acc_p + p
                acc_sq = p * p if acc_sq is None else acc_sq + p * p
            s0 = jnp.sum(acc_p, axis=0, keepdims=True) + float(MNB) * jnp.sum(
                u, axis=0, keepdims=True)
            s1 = (
                jnp.sum(acc_sq, axis=0, keepdims=True)
                + 2.0 * jnp.sum(acc_p * u, axis=0, keepdims=True)
                + float(MNB) * jnp.sum(u * u, axis=0, keepdims=True)
            )
            vals = jnp.concatenate([s0, s1], axis=0)

            @pl.when(i == 0)
            def _():
                st1[...] = vals

            @pl.when(i > 0)
            def _():
                st1[...] += vals

        @pl.when(ph == 1)
        def _():
            cnt = float(NEDGE)
            mean = st1[0:1, :] / cnt
            var = st1[1:2, :] / cnt - mean * mean
            scale = g1_ref[...] * lax.rsqrt(var + 1e-5)
            shift = b1_ref[...] - mean * scale
            wan_s = (wan_ref[...] * scale).astype(jnp.bfloat16)
            wnbr_s = (wnbr_ref[...] * scale).astype(jnp.bfloat16)
            wse_s = (wse_ref[...] * scale).astype(jnp.bfloat16)
            wan_f = wan_s[:, :AFD]
            wan_c = wan_s[:, AFD:]
            wnbr_f = wnbr_s[:, :AFD]
            wnbr_c = wnbr_s[:, AFD:]
            u = jnp.dot(a_ref[...].astype(jnp.bfloat16), wse_s,
                        preferred_element_type=jnp.float32) + shift
            uf = u[:, :AFD]
            uc = u[:, AFD:]
            sacc = None
            for m in range(MNB):
                ge = _unpack_bf(g_ref[m])
                nbm = n_ref[m]
                zf = (jnp.dot(ge, wan_f, preferred_element_type=jnp.float32)
                      + jnp.dot(nbm, wnbr_f, preferred_element_type=jnp.float32)
                      + uf)
                zc = (jnp.dot(ge, wan_c, preferred_element_type=jnp.float32)
                      + jnp.dot(nbm, wnbr_c, preferred_element_type=jnp.float32)
                      + uc)
                filt = 1.0 / (1.0 + jnp.exp(-zf))
                core = jnp.maximum(zc, 0.0) + jnp.log1p(jnp.exp(-jnp.abs(zc)))
                fc = filt * core
                sacc = fc if sacc is None else sacc + fc
            s_sc[pl.ds(i * _T1, _T1), :] = sacc
            v0 = jnp.sum(sacc, axis=0, keepdims=True)
            v1 = jnp.sum(sacc * sacc, axis=0, keepdims=True)
            vals = jnp.concatenate([v0, v1], axis=0)

            @pl.when(i == 0)
            def _():
                st2[...] = vals

            @pl.when(i > 0)
            def _():
                st2[...] += vals

        @pl.when(ph == 2)
        def _():
            cnt = float(NAT)
            mean = st2[0:1, :] / cnt
            var = st2[1:2, :] / cnt - mean * mean
            scale = g2_ref[...] * lax.rsqrt(var + 1e-5)
            shift = b2_ref[...] - mean * scale
            s_t = s_sc[pl.ds(i * _T1, _T1), :]
            zr = a_ref[...] + s_t * scale + shift
            an = jnp.maximum(zr, 0.0) + jnp.log1p(jnp.exp(-jnp.abs(zr)))
            o_ref[...] = an
            ab = an.astype(jnp.bfloat16)
            lo = lax.bitcast_convert_type(ab[:, : AFD // 2], jnp.uint16).astype(
                jnp.uint32)
            hi = lax.bitcast_convert_type(ab[:, AFD // 2 :], jnp.uint16).astype(
                jnp.uint32)
            pk_ref[...] = lax.bitcast_convert_type(lo | (hi << 16), jnp.int32)

    return pl.pallas_call(
        body,
        out_shape=[
            jax.ShapeDtypeStruct((NAT, AFD), jnp.float32),
            jax.ShapeDtypeStruct((NAT, AFD // 2), jnp.int32),
        ],
        grid=grid,
        in_specs=[
            pl.BlockSpec((_T1, AFD), lambda ph, i: (i, 0)),
            pl.BlockSpec((MNB, _T1, AFD // 2),
                         lambda ph, i: (0, jnp.where(ph < 2, i, 0), 0)),
            pl.BlockSpec((MNB, _T1, NBRF),
                         lambda ph, i: (0, jnp.where(ph < 2, i, 0), 0)),
            pl.BlockSpec((AFD, GFD), lambda ph, i: (0, 0)),
            pl.BlockSpec((AFD, GFD), lambda ph, i: (0, 0)),
            pl.BlockSpec((NBRF, GFD), lambda ph, i: (0, 0)),
            pl.BlockSpec((1, GFD), lambda ph, i: (0, 0)),
            pl.BlockSpec((1, GFD), lambda ph, i: (0, 0)),
            pl.BlockSpec((1, AFD), lambda ph, i: (0, 0)),
            pl.BlockSpec((1, AFD), lambda ph, i: (0, 0)),
        ],
        out_specs=[
            pl.BlockSpec((_T1, AFD), lambda ph, i: (jnp.where(ph < 2, 0, i), 0)),
            pl.BlockSpec((_T1, AFD // 2),
                         lambda ph, i: (jnp.where(ph < 2, 0, i), 0)),
        ],
        scratch_shapes=[
            pltpu.VMEM((NAT, AFD), jnp.float32),
            pltpu.VMEM((2, GFD), jnp.float32),
            pltpu.VMEM((2, AFD), jnp.float32),
        ],
    )(a, G3, nbrT, Wse, Wan, Wnbr, g1.reshape(1, GFD), b1.reshape(1, GFD),
      g2.reshape(1, AFD), b2.reshape(1, AFD))


def _final(gp1, gp2, m1, m2, sv, Wfc1, bfc1, Wfc2, bfc2, Wfus, bfus, Wout, bout):
    FI = Wfus.shape[0]

    def body(
        gp1_ref, gp2_ref, m1_ref, m2_ref, sv_ref, wfc1_ref, bfc1_ref, wfc2_ref,
        bfc2_ref, wfus_ref, bfus_ref, wout_ref, bout_ref, o_ref,
    ):
        def pool(gp_ref, wfc_ref, bfc_ref):
            x = _unpack(gp_ref[...]).reshape(NCRY, KPAD, AFD)
            iot = lax.broadcasted_iota(jnp.int32, (NCRY, KPAD, AFD), 1)
            x = jnp.where(iot < KCRY, x, 0.0)
            pooled = jnp.sum(x, axis=1) / float(KCRY)
            return (
                jnp.dot(pooled, wfc_ref[...], preferred_element_type=jnp.float32)
                + bfc_ref[...]
            )

        e1 = pool(gp1_ref, wfc1_ref, bfc1_ref)
        e2 = pool(gp2_ref, wfc2_ref, bfc2_ref)
        fused = jnp.concatenate(
            [e1, e2, m1_ref[...], m2_ref[...], sv_ref[...]], axis=1
        )
        h = jnp.dot(fused, wfus_ref[...], preferred_element_type=jnp.float32)
        h = jnp.maximum(h + bfus_ref[...], 0.0)
        o_ref[...] = (
            jnp.dot(h, wout_ref[...], preferred_element_type=jnp.float32)
            + bout_ref[...]
        )

    H = Wfc1.shape[1]
    return pl.pallas_call(
        body,
        out_shape=jax.ShapeDtypeStruct((NCRY, 1), jnp.float32),
        in_specs=[pl.BlockSpec(x.shape, lambda: tuple(0 for _ in x.shape))
                  for x in (gp1, gp2, m1, m2, sv)]
        + [
            pl.BlockSpec((AFD, H), lambda: (0, 0)),
            pl.BlockSpec((1, H), lambda: (0, 0)),
            pl.BlockSpec((AFD, H), lambda: (0, 0)),
            pl.BlockSpec((1, H), lambda: (0, 0)),
            pl.BlockSpec((FI, FI), lambda: (0, 0)),
            pl.BlockSpec((1, FI), lambda: (0, 0)),
            pl.BlockSpec((FI, 1), lambda: (0, 0)),
            pl.BlockSpec((1, 1), lambda: (0, 0)),
        ],
        out_specs=pl.BlockSpec((NCRY, 1), lambda: (0, 0)),
    )(
        gp1, gp2, m1, m2, sv,
        Wfc1, bfc1.reshape(1, H), Wfc2, bfc2.reshape(1, H),
        Wfus, bfus.reshape(1, FI), Wout, bout.reshape(1, 1),
    )


def _pad_crys(crys_idx):
    return jnp.concatenate(
        [
            crys_idx.reshape(NCRY, KCRY),
            jnp.zeros((NCRY, KPAD - KCRY), crys_idx.dtype),
        ],
        axis=1,
    ).reshape(NCRY * KPAD).astype(jnp.int32)


def kernel(atom, nbr, idx, crys_idx, atom2, nbr2, idx2, crys_idx2, s_vector,
           l_vector, mono_target1, mono_target2, W_emb, b_emb, convW, convb,
           bn1g, bn1b, bn2g, bn2b, W_fc, b_fc, W_emb2, b_emb2, convW2, convb2,
           bn1g2, bn1b2, bn2g2, bn2b2, W_fc2, b_fc2, W_fus, b_fus, W_out, b_out):
    a1 = _embed(atom, W_emb, b_emb)
    a2 = _embed(atom2, W_emb2, b_emb2)
    t1 = _pack_table(a1)
    t2 = _pack_table(a2)
    idx1 = idx.T.reshape(NEDGE).astype(jnp.int32)
    idx2f = idx2.T.reshape(NEDGE).astype(jnp.int32)
    nbrT1 = nbr.transpose(1, 0, 2).astype(jnp.bfloat16)
    nbrT2 = nbr2.transpose(1, 0, 2).astype(jnp.bfloat16)
    NC = convW.shape[0]
    G1 = _sc_gather(t1, idx1, chunk=200).reshape(MNB, NAT, AFD // 2)
    G2 = _sc_gather(t2, idx2f, chunk=200).reshape(MNB, NAT, AFD // 2)
    gp1 = gp2 = None
    for i in range(NC):
        a1, t1 = _conv_layer(a1, G1, nbrT1, convW[i, :AFD],
                             convW[i, AFD : 2 * AFD], convW[i, 2 * AFD :],
                             bn1g[i], bn1b[i], bn2g[i], bn2b[i])
        if i + 1 < NC:
            G1 = _sc_gather(t1, idx1, chunk=200).reshape(MNB, NAT, AFD // 2)
        else:
            gp1 = _sc_gather(t1, _pad_crys(crys_idx), chunk=320)
        a2, t2 = _conv_layer(a2, G2, nbrT2, convW2[i, :AFD],
                             convW2[i, AFD : 2 * AFD], convW2[i, 2 * AFD :],
                             bn1g2[i], bn1b2[i], bn2g2[i], bn2b2[i])
        if i + 1 < NC:
            G2 = _sc_gather(t2, idx2f, chunk=200).reshape(MNB, NAT, AFD // 2)
        else:
            gp2 = _sc_gather(t2, _pad_crys(crys_idx2), chunk=320)
    return _final(gp1, gp2, mono_target1, mono_target2, s_vector, W_fc, b_fc,
                  W_fc2, b_fc2, W_fus, b_fus, W_out, b_out)


# speedup vs baseline: 1.0588x; 1.0588x over previous
import functools

import jax
import jax.numpy as jnp
from jax import lax
from jax.experimental import pallas as pl
from jax.experimental.pallas import tpu as pltpu
from jax.experimental.pallas import tpu_sc as plsc

AFD = 128
GFD = 256
NBRF = 41
MNB = 16
NAT = 10000
NEDGE = NAT * MNB
NCRY = 16
KCRY = 625
KPAD = 640

_SC_CORES = 2
_SC_SUBCORES = 16
_NW = _SC_CORES * _SC_SUBCORES


def _sc_gather(table, idx_flat, chunk):
    B = idx_flat.shape[0]
    D = table.shape[1]
    cpw = B // (chunk * _NW)
    mesh = plsc.VectorSubcoreMesh(core_axis_name="c", subcore_axis_name="s")

    @functools.partial(
        pl.kernel,
        mesh=mesh,
        out_type=jax.ShapeDtypeStruct((B, D), table.dtype),
        compiler_params=pltpu.CompilerParams(use_tc_tiling_on_sc=False),
        scratch_types=[
            pltpu.VMEM((cpw * chunk,), jnp.int32),
            pltpu.VMEM((chunk, D), table.dtype),
            pltpu.VMEM((chunk, D), table.dtype),
            pltpu.SemaphoreType.DMA,
            pltpu.SemaphoreType.DMA,
        ],
    )
    def k(table_hbm, idx_hbm, out_hbm, idx_v, r0, r1, s0, s1):
        wid = lax.axis_index("s") * _SC_CORES + lax.axis_index("c")
        base = wid * cpw * chunk
        pltpu.sync_copy(idx_hbm.at[pl.ds(base, cpw * chunk)], idx_v)
        pltpu.async_copy(table_hbm.at[idx_v.at[pl.ds(0, chunk)]], r0, s0)

        def wait_rows(buf, sem):
            pltpu.make_async_copy(table_hbm.at[pl.ds(0, chunk)], buf, sem).wait()

        @pl.loop(0, cpw, step=2)
        def _(c):
            @pl.when(c + 1 < cpw)
            def _():
                pltpu.async_copy(
                    table_hbm.at[idx_v.at[pl.ds((c + 1) * chunk, chunk)]], r1, s1
                )

            wait_rows(r0, s0)
            pltpu.sync_copy(r0, out_hbm.at[pl.ds(base + c * chunk, chunk)])

            @pl.when(c + 2 < cpw)
            def _():
                pltpu.async_copy(
                    table_hbm.at[idx_v.at[pl.ds((c + 2) * chunk, chunk)]], r0, s0
                )

            @pl.when(c + 1 < cpw)
            def _():
                wait_rows(r1, s1)
                pltpu.sync_copy(
                    r1, out_hbm.at[pl.ds(base + (c + 1) * chunk, chunk)]
                )

    return k(table, idx_flat)


def _pack_table(a):
    ab = a.astype(jnp.bfloat16)
    lo = lax.bitcast_convert_type(ab[:, : AFD // 2], jnp.uint16).astype(jnp.uint32)
    hi = lax.bitcast_convert_type(ab[:, AFD // 2 :], jnp.uint16).astype(jnp.uint32)
    return lax.bitcast_convert_type(lo | (hi << 16), jnp.int32)


def _unpack(g32):
    lo = lax.bitcast_convert_type(g32 << 16, jnp.float32)
    hi = lax.bitcast_convert_type(g32 & jnp.int32(-65536), jnp.float32)
    return jnp.concatenate([lo, hi], axis=-1)


def _unpack_bf(g32):
    lo = lax.bitcast_convert_type(g32 << 16, jnp.float32).astype(jnp.bfloat16)
    hi = lax.bitcast_convert_type(
        g32 & jnp.int32(-65536), jnp.float32).astype(jnp.bfloat16)
    return jnp.concatenate([lo, hi], axis=-1)


def _embed(x, W, b):
    N, F = x.shape
    TO = W.shape[1]
    TN = 2000

    def body(x_ref, w_ref, b_ref, o_ref):
        o_ref[...] = (
            jnp.dot(x_ref[...], w_ref[...], preferred_element_type=jnp.float32)
            + b_ref[...]
        )

    return pl.pallas_call(
        body,
        out_shape=jax.ShapeDtypeStruct((N, TO), jnp.float32),
        grid=(N // TN,),
        in_specs=[
            pl.BlockSpec((TN, F), lambda i: (i, 0)),
            pl.BlockSpec((F, TO), lambda i: (0, 0)),
            pl.BlockSpec((1, TO), lambda i: (0, 0)),
        ],
        out_specs=pl.BlockSpec((TN, TO), lambda i: (i, 0)),
    )(x, W, b.reshape(1, TO))


_T1 = 1000


def _conv_layer(a, G3, nbrT, Wse, Wan, Wnbr, g1, b1, g2, b2):
    grid = (3, NAT // _T1)

    def body(a_ref, g_ref, n_ref, wse_ref, wan_ref, wnbr_ref, g1_ref, b1_ref,
             g2_ref, b2_ref, o_ref, pk_ref, s_sc, st1, st2):
        ph = pl.program_id(0)
        i = pl.program_id(1)

        @pl.when(ph == 0)
        def _():
            wan_bf = wan_ref[...].astype(jnp.bfloat16)
            wnbr_bf = wnbr_ref[...].astype(jnp.bfloat16)
            wse_bf = wse_ref[...].astype(jnp.bfloat16)
            u = jnp.dot(a_ref[...].astype(jnp.bfloat16), wse_bf,
                        preferred_element_type=jnp.float32)
            acc_p = None
            acc_sq = None
            for m in range(MNB):
                ge = _unpack_bf(g_ref[m])
                p = jnp.dot(ge, wan_bf, preferred_element_type=jnp.float32)
                p = p + jnp.dot(n_ref[m], wnbr_bf,
                                preferred_element_type=jnp.float32)
                acc_p = p if acc_p is None else acc_p + p
                acc_sq = p * p if acc_sq is None else acc_sq + p * p
            s0 = jnp.sum(acc_p, axis=0, keepdims=True) + float(MNB) * jnp.sum(
                u, axis=0, keepdims=True)
            s1 = (
                jnp.sum(acc_sq, axis=0, keepdims=True)
                + 2.0 * jnp.sum(acc_p * u, axis=0, keepdims=True)
                + float(MNB) * jnp.sum(u * u, axis=0, keepdims=True)
            )
            vals = jnp.concatenate([s0, s1], axis=0)

            @pl.when(i == 0)
            def _():
                st1[...] = vals

            @pl.when(i > 0)
            def _():
                st1[...] += vals

        @pl.when(ph == 1)
        def _():
            cnt = float(NEDGE)
            mean = st1[0:1, :] / cnt
            var = st1[1:2, :] / cnt - mean * mean
            scale = g1_ref[...] * lax.rsqrt(var + 1e-5)
            shift = b1_ref[...] - mean * scale
            wan_s = (wan_ref[...] * scale).astype(jnp.bfloat16)
            wnbr_s = (wnbr_ref[...] * scale).astype(jnp.bfloat16)
            wse_s = (wse_ref[...] * scale).astype(jnp.bfloat16)
            u = jnp.dot(a_ref[...].astype(jnp.bfloat16), wse_s,
                        preferred_element_type=jnp.float32) + shift
            sacc = None
            for m in range(MNB):
                ge = _unpack_bf(g_ref[m])
                z = jnp.dot(ge, wan_s, preferred_element_type=jnp.float32)
                z = z + jnp.dot(n_ref[m], wnbr_s,
                                preferred_element_type=jnp.float32)
                z = z + u
                filt = 1.0 / (1.0 + jnp.exp(-z[:, :AFD]))
                zc = z[:, AFD:]
                core = jnp.maximum(zc, 0.0) + jnp.log1p(jnp.exp(-jnp.abs(zc)))
                fc = filt * core
                sacc = fc if sacc is None else sacc + fc
            s_sc[pl.ds(i * _T1, _T1), :] = sacc
            v0 = jnp.sum(sacc, axis=0, keepdims=True)
            v1 = jnp.sum(sacc * sacc, axis=0, keepdims=True)
            vals = jnp.concatenate([v0, v1], axis=0)

            @pl.when(i == 0)
            def _():
                st2[...] = vals

            @pl.when(i > 0)
            def _():
                st2[...] += vals

        @pl.when(ph == 2)
        def _():
            cnt = float(NAT)
            mean = st2[0:1, :] / cnt
            var = st2[1:2, :] / cnt - mean * mean
            scale = g2_ref[...] * lax.rsqrt(var + 1e-5)
            shift = b2_ref[...] - mean * scale
            s_t = s_sc[pl.ds(i * _T1, _T1), :]
            zr = a_ref[...] + s_t * scale + shift
            an = jnp.maximum(zr, 0.0) + jnp.log1p(jnp.exp(-jnp.abs(zr)))
            o_ref[...] = an
            ab = an.astype(jnp.bfloat16)
            lo = lax.bitcast_convert_type(ab[:, : AFD // 2], jnp.uint16).astype(
                jnp.uint32)
            hi = lax.bitcast_convert_type(ab[:, AFD // 2 :], jnp.uint16).astype(
                jnp.uint32)
            pk_ref[...] = lax.bitcast_convert_type(lo | (hi << 16), jnp.int32)

    return pl.pallas_call(
        body,
        out_shape=[
            jax.ShapeDtypeStruct((NAT, AFD), jnp.float32),
            jax.ShapeDtypeStruct((NAT, AFD // 2), jnp.int32),
        ],
        grid=grid,
        in_specs=[
            pl.BlockSpec((_T1, AFD), lambda ph, i: (i, 0)),
            pl.BlockSpec((MNB, _T1, AFD // 2),
                         lambda ph, i: (0, jnp.where(ph < 2, i, 0), 0)),
            pl.BlockSpec((MNB, _T1, NBRF),
                         lambda ph, i: (0, jnp.where(ph < 2, i, 0), 0)),
            pl.BlockSpec((AFD, GFD), lambda ph, i: (0, 0)),
            pl.BlockSpec((AFD, GFD), lambda ph, i: (0, 0)),
            pl.BlockSpec((NBRF, GFD), lambda ph, i: (0, 0)),
            pl.BlockSpec((1, GFD), lambda ph, i: (0, 0)),
            pl.BlockSpec((1, GFD), lambda ph, i: (0, 0)),
            pl.BlockSpec((1, AFD), lambda ph, i: (0, 0)),
            pl.BlockSpec((1, AFD), lambda ph, i: (0, 0)),
        ],
        out_specs=[
            pl.BlockSpec((_T1, AFD), lambda ph, i: (jnp.where(ph < 2, 0, i), 0)),
            pl.BlockSpec((_T1, AFD // 2),
                         lambda ph, i: (jnp.where(ph < 2, 0, i), 0)),
        ],
        scratch_shapes=[
            pltpu.VMEM((NAT, AFD), jnp.float32),
            pltpu.VMEM((2, GFD), jnp.float32),
            pltpu.VMEM((2, AFD), jnp.float32),
        ],
    )(a, G3, nbrT, Wse, Wan, Wnbr, g1.reshape(1, GFD), b1.reshape(1, GFD),
      g2.reshape(1, AFD), b2.reshape(1, AFD))


def _final(gp1, gp2, m1, m2, sv, Wfc1, bfc1, Wfc2, bfc2, Wfus, bfus, Wout, bout):
    FI = Wfus.shape[0]

    def body(
        gp1_ref, gp2_ref, m1_ref, m2_ref, sv_ref, wfc1_ref, bfc1_ref, wfc2_ref,
        bfc2_ref, wfus_ref, bfus_ref, wout_ref, bout_ref, o_ref,
    ):
        def pool(gp_ref, wfc_ref, bfc_ref):
            x = _unpack(gp_ref[...]).reshape(NCRY, KPAD, AFD)
            iot = lax.broadcasted_iota(jnp.int32, (NCRY, KPAD, AFD), 1)
            x = jnp.where(iot < KCRY, x, 0.0)
            pooled = jnp.sum(x, axis=1) / float(KCRY)
            return (
                jnp.dot(pooled, wfc_ref[...], preferred_element_type=jnp.float32)
                + bfc_ref[...]
            )

        e1 = pool(gp1_ref, wfc1_ref, bfc1_ref)
        e2 = pool(gp2_ref, wfc2_ref, bfc2_ref)
        fused = jnp.concatenate(
            [e1, e2, m1_ref[...], m2_ref[...], sv_ref[...]], axis=1
        )
        h = jnp.dot(fused, wfus_ref[...], preferred_element_type=jnp.float32)
        h = jnp.maximum(h + bfus_ref[...], 0.0)
        o_ref[...] = (
            jnp.dot(h, wout_ref[...], preferred_element_type=jnp.float32)
            + bout_ref[...]
        )

    H = Wfc1.shape[1]
    return pl.pallas_call(
        body,
        out_shape=jax.ShapeDtypeStruct((NCRY, 1), jnp.float32),
        in_specs=[pl.BlockSpec(x.shape, lambda: tuple(0 for _ in x.shape))
                  for x in (gp1, gp2, m1, m2, sv)]
        + [
            pl.BlockSpec((AFD, H), lambda: (0, 0)),
            pl.BlockSpec((1, H), lambda: (0, 0)),
            pl.BlockSpec((AFD, H), lambda: (0, 0)),
            pl.BlockSpec((1, H), lambda: (0, 0)),
            pl.BlockSpec((FI, FI), lambda: (0, 0)),
            pl.BlockSpec((1, FI), lambda: (0, 0)),
            pl.BlockSpec((FI, 1), lambda: (0, 0)),
            pl.BlockSpec((1, 1), lambda: (0, 0)),
        ],
        out_specs=pl.BlockSpec((NCRY, 1), lambda: (0, 0)),
    )(
        gp1, gp2, m1, m2, sv,
        Wfc1, bfc1.reshape(1, H), Wfc2, bfc2.reshape(1, H),
        Wfus, bfus.reshape(1, FI), Wout, bout.reshape(1, 1),
    )


def _pad_crys(crys_idx):
    return jnp.concatenate(
        [
            crys_idx.reshape(NCRY, KCRY),
            jnp.zeros((NCRY, KPAD - KCRY), crys_idx.dtype),
        ],
        axis=1,
    ).reshape(NCRY * KPAD).astype(jnp.int32)


def kernel(atom, nbr, idx, crys_idx, atom2, nbr2, idx2, crys_idx2, s_vector,
           l_vector, mono_target1, mono_target2, W_emb, b_emb, convW, convb,
           bn1g, bn1b, bn2g, bn2b, W_fc, b_fc, W_emb2, b_emb2, convW2, convb2,
           bn1g2, bn1b2, bn2g2, bn2b2, W_fc2, b_fc2, W_fus, b_fus, W_out, b_out):
    a1 = _embed(atom, W_emb, b_emb)
    a2 = _embed(atom2, W_emb2, b_emb2)
    t1 = _pack_table(a1)
    t2 = _pack_table(a2)
    idx1 = idx.T.reshape(NEDGE).astype(jnp.int32)
    idx2f = idx2.T.reshape(NEDGE).astype(jnp.int32)
    nbrT1 = nbr.transpose(1, 0, 2).astype(jnp.bfloat16)
    nbrT2 = nbr2.transpose(1, 0, 2).astype(jnp.bfloat16)
    NC = convW.shape[0]
    G1 = _sc_gather(t1, idx1, chunk=200).reshape(MNB, NAT, AFD // 2)
    G2 = _sc_gather(t2, idx2f, chunk=200).reshape(MNB, NAT, AFD // 2)
    gp1 = gp2 = None
    for i in range(NC):
        a1, t1 = _conv_layer(a1, G1, nbrT1, convW[i, :AFD],
                             convW[i, AFD : 2 * AFD], convW[i, 2 * AFD :],
                             bn1g[i], bn1b[i], bn2g[i], bn2b[i])
        if i + 1 < NC:
            G1 = _sc_gather(t1, idx1, chunk=200).reshape(MNB, NAT, AFD // 2)
        else:
            gp1 = _sc_gather(t1, _pad_crys(crys_idx), chunk=320)
        a2, t2 = _conv_layer(a2, G2, nbrT2, convW2[i, :AFD],
                             convW2[i, AFD : 2 * AFD], convW2[i, 2 * AFD :],
                             bn1g2[i], bn1b2[i], bn2g2[i], bn2b2[i])
        if i + 1 < NC:
            G2 = _sc_gather(t2, idx2f, chunk=200).reshape(MNB, NAT, AFD // 2)
        else:
            gp2 = _sc_gather(t2, _pad_crys(crys_idx2), chunk=320)
    return _final(gp1, gp2, mono_target1, mono_target2, s_vector, W_fc, b_fc,
                  W_fc2, b_fc2, W_fus, b_fus, W_out, b_out)
